# baseline (speedup 1.0000x reference)
import functools

import jax
import jax.numpy as jnp
from jax import lax
from jax.experimental import pallas as pl
from jax.experimental.pallas import tpu as pltpu
from jax.experimental.pallas import tpu_sc as plsc

BAG = 16384
EMBED_DIM = 64
HIDDEN = 128
NC, NS, L = 2, 16, 16
NW = NC * NS
B_PER_W = BAG // NW
CHUNK = 128
N_CHUNKS = B_PER_W // CHUNK
UNROLL = 8

_mesh = plsc.VectorSubcoreMesh(core_axis_name="c", subcore_axis_name="s")


@functools.partial(
    pl.kernel,
    mesh=_mesh,
    out_type=jax.ShapeDtypeStruct((NW, EMBED_DIM), jnp.float32),
    scratch_types=[
        pltpu.VMEM((N_CHUNKS, CHUNK), jnp.int32),
        pltpu.VMEM((B_PER_W, EMBED_DIM), jnp.float32),
        pltpu.VMEM((EMBED_DIM,), jnp.float32),
        pltpu.SemaphoreType.DMA,
    ],
    compiler_params=pltpu.CompilerParams(use_tc_tiling_on_sc=False),
)
def _sc_gather_sum(idx_hbm, table_hbm, out_hbm, idx_v, rows_v, acc_v, sem):
    wid = lax.axis_index("s") * NC + lax.axis_index("c")
    pltpu.sync_copy(idx_hbm.at[wid], idx_v)
    copies = [
        pltpu.async_copy(
            table_hbm.at[idx_v.at[j]],
            rows_v.at[pl.ds(j * CHUNK, CHUNK)],
            sem,
        )
        for j in range(N_CHUNKS)
    ]
    for c in copies:
        c.wait()

    def body(i, acc):
        out = []
        for k in range(EMBED_DIM // L):
            a = acc[k]
            for r in range(UNROLL):
                a = a + rows_v[i * UNROLL + r, pl.ds(k * L, L)]
            out.append(a)
        return tuple(out)

    zero = jnp.zeros((L,), jnp.float32)
    acc = lax.fori_loop(0, B_PER_W // UNROLL, body,
                        (zero,) * (EMBED_DIM // L))
    for k in range(EMBED_DIM // L):
        acc_v[pl.ds(k * L, L)] = acc[k]
    pltpu.sync_copy(acc_v, out_hbm.at[wid])


def _tc_head(partials_ref, hidden_ref, w1t_ref, b1_ref, w2t_ref, b2_ref,
             out_ref, hid_ref):
    emb = jnp.sum(partials_ref[...], axis=0, keepdims=True) * (1.0 / BAG)
    combined = jnp.concatenate([emb, hidden_ref[...]], axis=1)
    nh = jnp.dot(combined, w1t_ref[...],
                 preferred_element_type=jnp.float32) + b1_ref[...]
    hid_ref[...] = nh
    logit = jnp.dot(combined, w2t_ref[...],
                    preferred_element_type=jnp.float32) + b2_ref[...]
    out_ref[...] = 1.0 / (1.0 + jnp.exp(-logit))


_head = pl.pallas_call(
    _tc_head,
    out_shape=(
        jax.ShapeDtypeStruct((1, 1), jnp.float32),
        jax.ShapeDtypeStruct((1, HIDDEN), jnp.float32),
    ),
)


def kernel(input_, hidden, emb_table, W_i2h, b_i2h, W_i2o, b_i2o):
    idx_r = input_.reshape(NW, N_CHUNKS, CHUNK)
    partials = _sc_gather_sum(idx_r, emb_table)
    output, new_hidden = _head(
        partials, hidden,
        W_i2h.T, b_i2h.reshape(1, HIDDEN),
        W_i2o.T, b_i2o.reshape(1, 1),
    )
    return (output, new_hidden)


# speedup vs baseline: 2.9776x; 2.9776x over previous
import functools

import jax
import jax.numpy as jnp
from jax import lax
from jax.experimental import pallas as pl
from jax.experimental.pallas import tpu as pltpu
from jax.experimental.pallas import tpu_sc as plsc

BAG = 16384
EMBED_DIM = 64
HIDDEN = 128
NC, NS, L = 2, 16, 16
NW = NC * NS
B_PER_W = BAG // NW
NB = 8
PANEL = 128
NVEC = EMBED_DIM // L

_mesh = plsc.VectorSubcoreMesh(core_axis_name="c", subcore_axis_name="s")


@functools.partial(
    pl.kernel,
    mesh=_mesh,
    out_type=jax.ShapeDtypeStruct((NW * EMBED_DIM,), jnp.float32),
    scratch_types=[
        pltpu.VMEM((B_PER_W,), jnp.int32),
        pltpu.VMEM((NB, EMBED_DIM, PANEL), jnp.float32),
        pltpu.VMEM((EMBED_DIM,), jnp.float32),
    ] + [pltpu.SemaphoreType.DMA] * NB,
    compiler_params=pltpu.CompilerParams(use_tc_tiling_on_sc=True,
                                         needs_layout_passes=False),
)
def _sc_gather_sum(idx_hbm, tabt_hbm, out_hbm, idx_v, ring_v, acc_v, *sems):
    wid = lax.axis_index("s") * NC + lax.axis_index("c")
    pltpu.sync_copy(idx_hbm.at[pl.ds(wid * B_PER_W, B_PER_W)], idx_v)

    lane = jax.lax.iota(jnp.int32, L)

    def issue(idx, slot):
        tb = pl.multiple_of(jnp.bitwise_and(idx, -PANEL), PANEL)
        pltpu.async_copy(
            tabt_hbm.at[:, pl.ds(tb, PANEL)], ring_v.at[slot], sems[slot]
        )

    def wait_acc(idx, slot, accs):
        pltpu.make_async_copy(
            tabt_hbm.at[:, pl.ds(0, PANEL)], ring_v.at[slot], sems[slot]
        ).wait()
        off = jnp.full((L,), jnp.bitwise_and(idx, PANEL - 1), jnp.int32)
        slot_vec = jnp.full((L,), slot, jnp.int32)
        return tuple(
            accs[k] + plsc.load_gather(ring_v, [slot_vec, lane + k * L, off])
            for k in range(NVEC)
        )

    chunk0 = idx_v[pl.ds(0, L)]
    for s in range(NB):
        issue(chunk0[s], s)

    def body(g, carry):
        accs, cur = carry[:-1], carry[-1]
        nxt = idx_v[pl.ds((g + 1) * L, L)]
        for s in range(L):
            accs = wait_acc(cur[s], s % NB, accs)
            issue(cur[s + NB] if s < L - NB else nxt[s - (L - NB)], s % NB)
        return (*accs, nxt)

    zero = jnp.zeros((L,), jnp.float32)
    carry = lax.fori_loop(0, B_PER_W // L - 1, body,
                          (*(zero,) * NVEC, chunk0))
    accs, cur = carry[:-1], carry[-1]
    for s in range(L):
        accs = wait_acc(cur[s], s % NB, accs)
        if s < L - NB:
            issue(cur[s + NB], s % NB)

    for k in range(NVEC):
        acc_v[pl.ds(k * L, L)] = accs[k]
    pltpu.sync_copy(acc_v, out_hbm.at[pl.ds(wid * EMBED_DIM, EMBED_DIM)])


def _tc_head(partials_ref, hidden_ref, w1t_ref, b1_ref, w2t_ref, b2_ref,
             out_ref, hid_ref):
    emb = jnp.sum(partials_ref[...], axis=0, keepdims=True) * (1.0 / BAG)
    combined = jnp.concatenate([emb, hidden_ref[...]], axis=1)
    nh = jnp.dot(combined, w1t_ref[...],
                 preferred_element_type=jnp.float32) + b1_ref[...]
    hid_ref[...] = nh
    logit = jnp.dot(combined, w2t_ref[...],
                    preferred_element_type=jnp.float32) + b2_ref[...]
    out_ref[...] = 1.0 / (1.0 + jnp.exp(-logit))


_head = pl.pallas_call(
    _tc_head,
    out_shape=(
        jax.ShapeDtypeStruct((1, 1), jnp.float32),
        jax.ShapeDtypeStruct((1, HIDDEN), jnp.float32),
    ),
)


def kernel(input_, hidden, emb_table, W_i2h, b_i2h, W_i2o, b_i2o):
    partials = _sc_gather_sum(input_, emb_table.T).reshape(NW, EMBED_DIM)
    output, new_hidden = _head(
        partials, hidden,
        W_i2h.T, b_i2h.reshape(1, HIDDEN),
        W_i2o.T, b_i2o.reshape(1, 1),
    )
    return (output, new_hidden)
